# Initial kernel scaffold; baseline (speedup 1.0000x reference)
import jax
import jax.numpy as jnp
from jax.experimental import pallas as pl


def kernel(x, emb_table):
    raise NotImplementedError("write your pallas kernel here")


# speedup vs baseline: 2.2007x; 2.2007x over previous
import functools

import jax
import jax.numpy as jnp
from jax import lax
from jax.experimental import pallas as pl
from jax.experimental.pallas import tpu as pltpu
from jax.experimental.pallas import tpu_sc as plsc

NUM_HID = 64
BATCH = 1024
SEQ_LEN = 200

_NC = 2
_NS = 16
_NW = _NC * _NS
_SEQ_PER_W = BATCH // _NW


def _pos_encoding():
    positions = jnp.arange(SEQ_LEN, dtype=jnp.float32)[:, None]
    depth = NUM_HID / 2
    depths = jnp.arange(depth, dtype=jnp.float32)[None, :] / depth
    angle_rates = 1.0 / (10000.0 ** depths)
    angle_rads = positions * angle_rates
    return jnp.concatenate(
        [jnp.sin(angle_rads), jnp.cos(angle_rads)], axis=-1)


def _sc_body(x_hbm, tab_hbm, pe_hbm, out_hbm, idx_v, rows_v, sem):
    wid = lax.axis_index("s") * _NC + lax.axis_index("c")

    def one_seq(i, carry):
        base = (wid * _SEQ_PER_W + i) * SEQ_LEN
        pltpu.sync_copy(x_hbm.at[pl.ds(base, SEQ_LEN)], idx_v)
        pltpu.sync_copy(pe_hbm, rows_v)
        pltpu.async_copy(tab_hbm.at[idx_v], rows_v, sem, add=True).wait()
        pltpu.sync_copy(rows_v, out_hbm.at[pl.ds(base, SEQ_LEN)])
        return carry

    lax.fori_loop(0, _SEQ_PER_W, one_seq, 0)


@jax.jit
def _run(x_flat, emb_table, pe):
    mesh = plsc.VectorSubcoreMesh(
        core_axis_name="c", subcore_axis_name="s",
        num_cores=_NC, num_subcores=_NS)
    kern = functools.partial(
        pl.kernel,
        out_type=jax.ShapeDtypeStruct((BATCH * SEQ_LEN, NUM_HID), jnp.float32),
        mesh=mesh,
        scratch_types=[
            pltpu.VMEM((SEQ_LEN,), jnp.int32),
            pltpu.VMEM((SEQ_LEN, NUM_HID), jnp.float32),
            pltpu.SemaphoreType.DMA,
        ],
        compiler_params=pltpu.CompilerParams(use_tc_tiling_on_sc=False),
    )(_sc_body)
    return kern(x_flat, emb_table, pe)


def kernel(x, emb_table):
    pe = _pos_encoding()
    x_flat = x.reshape(-1).astype(jnp.int32)
    out = _run(x_flat, emb_table, pe)
    return out.reshape(BATCH, SEQ_LEN, NUM_HID)


# speedup vs baseline: 3.0679x; 1.3940x over previous
import functools

import jax
import jax.numpy as jnp
from jax import lax
from jax.experimental import pallas as pl
from jax.experimental.pallas import tpu as pltpu
from jax.experimental.pallas import tpu_sc as plsc

NUM_HID = 64
BATCH = 1024
SEQ_LEN = 200

_NC = 2
_NS = 16
_NW = _NC * _NS
_SEQ_PER_W = BATCH // _NW
_CHUNK = 4
_NCHUNK = _SEQ_PER_W // _CHUNK
_ROWS = _CHUNK * SEQ_LEN


def _pos_encoding():
    positions = jnp.arange(SEQ_LEN, dtype=jnp.float32)[:, None]
    depth = NUM_HID / 2
    depths = jnp.arange(depth, dtype=jnp.float32)[None, :] / depth
    angle_rates = 1.0 / (10000.0 ** depths)
    angle_rads = positions * angle_rates
    return jnp.concatenate(
        [jnp.sin(angle_rads), jnp.cos(angle_rads)], axis=-1)


def _sc_body(x_hbm, tab_hbm, pe_hbm, out_hbm,
             idx0, idx1, rows0, rows1, pe_sh,
             sem_g0, sem_g1, sem_s0, sem_s1):
    c = lax.axis_index("c")
    s = lax.axis_index("s")
    wid = s * _NC + c

    @pl.when(s == 0)
    def _stage():
        pltpu.sync_copy(pe_hbm, rows0.at[pl.ds(0, SEQ_LEN)])
        for k in range(_CHUNK):
            pltpu.sync_copy(rows0.at[pl.ds(0, SEQ_LEN)],
                            pe_sh.at[pl.ds(k * SEQ_LEN, SEQ_LEN)])
    plsc.subcore_barrier()

    idxs = (idx0, idx1)
    rows = (rows0, rows1)
    sem_g = (sem_g0, sem_g1)
    sem_s = (sem_s0, sem_s1)
    gather_d = [None, None]
    scatter_d = [None, None]
    base_w = wid * _SEQ_PER_W * SEQ_LEN

    for g in range(_NCHUNK):
        b = g & 1
        base = base_w + g * _ROWS
        if scatter_d[b] is not None:
            scatter_d[b].wait()
        pltpu.sync_copy(x_hbm.at[pl.ds(base, _ROWS)], idxs[b])
        pltpu.sync_copy(pe_sh, rows[b])
        gather_d[b] = pltpu.async_copy(
            tab_hbm.at[idxs[b]], rows[b], sem_g[b], add=True)
        if g > 0:
            pb = 1 - b
            gather_d[pb].wait()
            pbase = base_w + (g - 1) * _ROWS
            scatter_d[pb] = pltpu.async_copy(
                rows[pb], out_hbm.at[pl.ds(pbase, _ROWS)], sem_s[pb])

    last = (_NCHUNK - 1) & 1
    gather_d[last].wait()
    lbase = base_w + (_NCHUNK - 1) * _ROWS
    scatter_d[last] = pltpu.async_copy(
        rows[last], out_hbm.at[pl.ds(lbase, _ROWS)], sem_s[last])
    scatter_d[1 - last].wait()
    scatter_d[last].wait()


@jax.jit
def _run(x_flat, emb_table, pe):
    mesh = plsc.VectorSubcoreMesh(
        core_axis_name="c", subcore_axis_name="s",
        num_cores=_NC, num_subcores=_NS)
    kern = functools.partial(
        pl.kernel,
        out_type=jax.ShapeDtypeStruct((BATCH * SEQ_LEN, NUM_HID), jnp.float32),
        mesh=mesh,
        scratch_types=[
            pltpu.VMEM((_ROWS,), jnp.int32),
            pltpu.VMEM((_ROWS,), jnp.int32),
            pltpu.VMEM((_ROWS, NUM_HID), jnp.float32),
            pltpu.VMEM((_ROWS, NUM_HID), jnp.float32),
            pltpu.VMEM_SHARED((_ROWS, NUM_HID), jnp.float32),
            pltpu.SemaphoreType.DMA,
            pltpu.SemaphoreType.DMA,
            pltpu.SemaphoreType.DMA,
            pltpu.SemaphoreType.DMA,
        ],
        compiler_params=pltpu.CompilerParams(use_tc_tiling_on_sc=False),
    )(_sc_body)
    return kern(x_flat, emb_table, pe)


def kernel(x, emb_table):
    pe = _pos_encoding()
    x_flat = x.reshape(-1).astype(jnp.int32)
    out = _run(x_flat, emb_table, pe)
    return out.reshape(BATCH, SEQ_LEN, NUM_HID)
